# baseline (speedup 1.0000x reference)
import functools

import jax
import jax.numpy as jnp
from jax import lax
from jax.experimental import pallas as pl
from jax.experimental.pallas import tpu as pltpu

_DIMS_T = (((1,), (1,)), ((), ()))


def _hilo(x):
    hi = x.astype(jnp.bfloat16)
    lo = (x - hi.astype(jnp.float32)).astype(jnp.bfloat16)
    return jnp.concatenate([hi, lo], axis=0)


def _cent_kernel(slic_ref, src_ref, tar_ref, oa_ref, ov_ref, *, s):
    k = pl.program_id(1)

    @pl.when(k == 0)
    def _():
        oa_ref[...] = jnp.zeros_like(oa_ref)
        ov_ref[...] = jnp.zeros_like(ov_ref)

    slic = slic_ref[0]
    src = src_ref[0]
    tar = tar_ref[0]
    tp = src.shape[1]
    tar_neg = (tar[0:1] < 0.0) | (tar[1:2] < 0.0)
    seg_v = jnp.where(slic < 0, s, slic)
    seg_a = jnp.where(tar_neg, s, seg_v)
    iota = lax.broadcasted_iota(jnp.int32, (s + 1, tp), 0)
    oh_a = (seg_a == iota).astype(jnp.bfloat16)
    oh_v = (seg_v == iota).astype(jnp.bfloat16)
    ones = jnp.ones((1, tp), jnp.float32)
    data_a = _hilo(jnp.concatenate([src, tar, ones], axis=0))
    data_v = _hilo(jnp.concatenate([src, ones], axis=0))
    ra = lax.dot_general(data_a, oh_a, _DIMS_T, preferred_element_type=jnp.float32)
    rv = lax.dot_general(data_v, oh_v, _DIMS_T, preferred_element_type=jnp.float32)
    oa_ref[0] += ra[:5] + ra[5:]
    ov_ref[0] += rv[:3] + rv[3:]


def _unet_kernel(slic_ref, src_ref, tar_ref, msk_ref, aux_ref,
                 wfg_ref, bfg_ref, wo_ref, bo_ref, o_ref, *, s, chn, rot):
    k = pl.program_id(1)

    @pl.when(k == 0)
    def _():
        o_ref[...] = jnp.zeros_like(o_ref)

    slic = slic_ref[0]
    src = src_ref[0]
    tar = tar_ref[0]
    tp = src.shape[1]
    ca = aux_ref.shape[1]
    tar_neg = (tar[0:1] < 0.0) | (tar[1:2] < 0.0)
    seg = jnp.where(tar_neg | (slic < 0), s, slic)
    iota = lax.broadcasted_iota(jnp.int32, (s + 1, tp), 0)
    oh = (seg == iota).astype(jnp.bfloat16)
    g2 = jnp.dot(_hilo(aux_ref[0]), oh, preferred_element_type=jnp.float32)
    g = g2[:ca] + g2[ca:]
    if rot:
        a = g[0:1]
        b = g[1:2]
        rx = src[0:1] * (1.0 + a) + src[1:2] * b
        ry = -src[0:1] * b + src[1:2] * (1.0 + a)
        pm = jnp.concatenate([rx, ry, tar], axis=0)
    else:
        pm = jnp.concatenate([src - g[:2], tar - g[2:4]], axis=0)
    valid = jnp.logical_not(tar_neg)
    pm = jnp.where(valid, pm, -1.0)
    x = jnp.concatenate([pm, msk_ref[0]], axis=0).astype(jnp.bfloat16)

    fg = jnp.dot(wfg_ref[...], x, preferred_element_type=jnp.float32) + bfg_ref[...]
    feat = jnp.maximum(fg[:chn], 0.0)
    gate = jax.nn.sigmoid(fg[chn:])
    h = (feat * gate).astype(jnp.bfloat16)
    out = jnp.dot(wo_ref[...], h, preferred_element_type=jnp.float32) + bo_ref[...]
    out = jnp.maximum(out, 0.0)
    o_ref[0] += lax.dot_general(out.astype(jnp.bfloat16), oh,
                                _DIMS_T, preferred_element_type=jnp.float32)


def _mreg_kernel(s_ref, c_ref, w0_ref, b0_ref, w1_ref, b1_ref, o_ref):
    feat = s_ref[0] / jnp.maximum(c_ref[0], 1.0)
    h = jnp.dot(w0_ref[...], feat.astype(jnp.bfloat16),
                preferred_element_type=jnp.float32) + b0_ref[...]
    h = jnp.maximum(h, 0.0)
    o_ref[0] = jnp.dot(w1_ref[...], h.astype(jnp.bfloat16),
                       preferred_element_type=jnp.float32) + b1_ref[...]


def _u_pre_kernel(x_ref, w1_ref, b1_ref, w2_ref, b2_ref, w3_ref, b3_ref,
                  o_ref, g_ref, *, s):
    x = x_ref[0].astype(jnp.bfloat16)
    h = jnp.maximum(jnp.dot(w1_ref[...], x,
                            preferred_element_type=jnp.float32) + b1_ref[...], 0.0)
    h = jnp.maximum(jnp.dot(w2_ref[...], h.astype(jnp.bfloat16),
                            preferred_element_type=jnp.float32) + b2_ref[...], 0.0)
    h = jnp.maximum(jnp.dot(w3_ref[...], h.astype(jnp.bfloat16),
                            preferred_element_type=jnp.float32) + b3_ref[...], 0.0)
    u = h.astype(jnp.bfloat16)
    o_ref[0] = u
    u3 = u.reshape(u.shape[0], s, s)
    g_ref[0] = jnp.concatenate([jnp.max(u3, axis=2), jnp.max(u3, axis=1)], axis=1)


def _u_global_kernel(x_ref, w1_ref, b1_ref, w2_ref, b2_ref, w3_ref, b3_ref,
                     wg0_ref, wg1_ref, o_ref, *, s):
    x = x_ref[0]
    h = jnp.maximum(jnp.dot(w1_ref[...], x,
                            preferred_element_type=jnp.float32) + b1_ref[...], 0.0)
    h = jnp.maximum(jnp.dot(w2_ref[...], h.astype(jnp.bfloat16),
                            preferred_element_type=jnp.float32) + b2_ref[...], 0.0)
    h = jnp.maximum(jnp.dot(w3_ref[...], h.astype(jnp.bfloat16),
                            preferred_element_type=jnp.float32) + b3_ref[...], 0.0)
    g = h.astype(jnp.bfloat16)
    pr = jnp.dot(wg0_ref[...], g[:, :s], preferred_element_type=jnp.float32)
    pc = jnp.dot(wg1_ref[...], g[:, s:], preferred_element_type=jnp.float32)
    o_ref[0] = jnp.concatenate([pr, pc], axis=1)


def _u_post_kernel(u_ref, pg_ref, wu_ref, b1_ref, w2_ref, b2_ref,
                   w3_ref, b3_ref, w4_ref, b4_ref, o_ref, *, s, tn):
    k = pl.program_id(1)
    p = k * tn + lax.broadcasted_iota(jnp.int32, (1, tn), 1)
    rid = p // s
    cid = p - rid * s
    riota = lax.broadcasted_iota(jnp.int32, (s, tn), 0)
    sel = jnp.concatenate([(rid == riota).astype(jnp.bfloat16),
                           (cid == riota).astype(jnp.bfloat16)], axis=0)
    h = jnp.dot(wu_ref[...], u_ref[0], preferred_element_type=jnp.float32)
    h = h + jnp.dot(pg_ref[0].astype(jnp.bfloat16), sel,
                    preferred_element_type=jnp.float32)
    h = jnp.maximum(h + b1_ref[...], 0.0)
    h = jnp.maximum(jnp.dot(w2_ref[...], h.astype(jnp.bfloat16),
                            preferred_element_type=jnp.float32) + b2_ref[...], 0.0)
    h = jnp.maximum(jnp.dot(w3_ref[...], h.astype(jnp.bfloat16),
                            preferred_element_type=jnp.float32) + b3_ref[...], 0.0)
    o_ref[0] = jnp.dot(w4_ref[...], h.astype(jnp.bfloat16),
                       preferred_element_type=jnp.float32) + b4_ref[...]


def _wT(w):
    return jnp.transpose(w).astype(jnp.bfloat16)


def _bc(b):
    return b.reshape(-1, 1).astype(jnp.float32)


def kernel(pos_src, pos_tar, mask, slic_map, src_pixel_group, dst_pixel_group,
           unet_r_feat_w, unet_r_feat_b, unet_r_gate_w, unet_r_gate_b,
           unet_r_out_w, unet_r_out_b,
           unet_t_feat_w, unet_t_feat_b, unet_t_gate_w, unet_t_gate_b,
           unet_t_out_w, unet_t_out_b,
           mreg_r_0_w, mreg_r_0_b, mreg_r_1_w, mreg_r_1_b,
           mreg_t_0_w, mreg_t_0_b, mreg_t_1_w, mreg_t_1_b,
           u_pre_0_w, u_pre_0_b, u_pre_1_w, u_pre_1_b, u_pre_2_w, u_pre_2_b,
           u_global_0_w, u_global_0_b, u_global_1_w, u_global_1_b,
           u_global_2_w, u_global_2_b,
           u_post_0_w, u_post_0_b, u_post_1_w, u_post_1_b,
           u_post_2_w, u_post_2_b, u_post_3_w, u_post_3_b):
    B, _, H, W = pos_src.shape
    P = H * W
    S = src_pixel_group.shape[1]
    S1 = S + 1
    SS = S * S
    f32 = jnp.float32

    src = pos_src.reshape(B, 2, P)
    tar = pos_tar.reshape(B, 2, P)
    msk = mask.reshape(B, 1, P)
    slic = slic_map.reshape(B, 1, P).astype(jnp.int32)

    nk = 2 if P % 2 == 0 else 1
    TP = P // nk
    par_arb = pltpu.CompilerParams(dimension_semantics=("parallel", "arbitrary"))
    par_par = pltpu.CompilerParams(dimension_semantics=("parallel", "parallel"))
    par = pltpu.CompilerParams(dimension_semantics=("parallel",))

    sums_a, sums_v = pl.pallas_call(
        functools.partial(_cent_kernel, s=S),
        out_shape=(jax.ShapeDtypeStruct((B, 5, S1), f32),
                   jax.ShapeDtypeStruct((B, 3, S1), f32)),
        grid=(B, nk),
        in_specs=[
            pl.BlockSpec((1, 1, TP), lambda i, k: (i, 0, k)),
            pl.BlockSpec((1, 2, TP), lambda i, k: (i, 0, k)),
            pl.BlockSpec((1, 2, TP), lambda i, k: (i, 0, k)),
        ],
        out_specs=(pl.BlockSpec((1, 5, S1), lambda i, k: (i, 0, 0)),
                   pl.BlockSpec((1, 3, S1), lambda i, k: (i, 0, 0))),
        compiler_params=par_arb,
    )(slic, src, tar)

    counts = sums_a[:, 4:5]
    cent = sums_a[:, :4] / jnp.maximum(counts, 1.0)
    pos_sp = (sums_v[:, :2] / jnp.maximum(sums_v[:, 2:3], 1.0))[:, :, :S]

    def unet_call(aux, fw, fb, gw, gb, ow, ob, rot):
        ca = aux.shape[1]
        wfg = jnp.transpose(jnp.concatenate([fw, gw], axis=1)).astype(jnp.bfloat16)
        bfg = jnp.concatenate([fb, gb]).reshape(-1, 1).astype(f32)
        wo = _wT(ow)
        bo = _bc(ob)
        chn = fw.shape[1]
        return pl.pallas_call(
            functools.partial(_unet_kernel, s=S, chn=chn, rot=rot),
            out_shape=jax.ShapeDtypeStruct((B, chn, S1), f32),
            grid=(B, nk),
            in_specs=[
                pl.BlockSpec((1, 1, TP), lambda i, k: (i, 0, k)),
                pl.BlockSpec((1, 2, TP), lambda i, k: (i, 0, k)),
                pl.BlockSpec((1, 2, TP), lambda i, k: (i, 0, k)),
                pl.BlockSpec((1, 1, TP), lambda i, k: (i, 0, k)),
                pl.BlockSpec((1, ca, S1), lambda i, k: (i, 0, 0)),
                pl.BlockSpec(wfg.shape, lambda i, k: (0, 0)),
                pl.BlockSpec(bfg.shape, lambda i, k: (0, 0)),
                pl.BlockSpec(wo.shape, lambda i, k: (0, 0)),
                pl.BlockSpec(bo.shape, lambda i, k: (0, 0)),
            ],
            out_specs=pl.BlockSpec((1, chn, S1), lambda i, k: (i, 0, 0)),
            compiler_params=par_arb,
        )(slic, src, tar, msk, aux, wfg, bfg, wo, bo)

    def mreg_call(sums, w0, b0, w1, b1):
        w0t, w1t = _wT(w0), _wT(w1)
        b0c, b1c = _bc(b0), _bc(b1)
        return pl.pallas_call(
            _mreg_kernel,
            out_shape=jax.ShapeDtypeStruct((B, 2, S1), f32),
            grid=(B,),
            in_specs=[
                pl.BlockSpec((1, 16, S1), lambda i: (i, 0, 0)),
                pl.BlockSpec((1, 1, S1), lambda i: (i, 0, 0)),
                pl.BlockSpec(w0t.shape, lambda i: (0, 0)),
                pl.BlockSpec(b0c.shape, lambda i: (0, 0)),
                pl.BlockSpec(w1t.shape, lambda i: (0, 0)),
                pl.BlockSpec(b1c.shape, lambda i: (0, 0)),
            ],
            out_specs=pl.BlockSpec((1, 2, S1), lambda i: (i, 0, 0)),
            compiler_params=par,
        )(sums, counts, w0t, b0c, w1t, b1c)

    sum_R = unet_call(cent, unet_r_feat_w, unet_r_feat_b, unet_r_gate_w,
                      unet_r_gate_b, unet_r_out_w, unet_r_out_b, rot=False)
    pred_ab = mreg_call(sum_R, mreg_r_0_w, mreg_r_0_b, mreg_r_1_w, mreg_r_1_b)

    a = pred_ab[:, 0, :S]
    b = pred_ab[:, 1, :S]
    pred_R = jnp.stack([jnp.stack([1.0 + a, -b], axis=-1),
                        jnp.stack([b, 1.0 + a], axis=-1)], axis=-2)

    sum_T = unet_call(pred_ab, unet_t_feat_w, unet_t_feat_b, unet_t_gate_w,
                      unet_t_gate_b, unet_t_out_w, unet_t_out_b, rot=True)
    pred_t_ab = mreg_call(sum_T, mreg_t_0_w, mreg_t_0_b, mreg_t_1_w, mreg_t_1_b)
    pred_T = jnp.transpose(pred_t_ab, (0, 2, 1))[:, :S][:, :, None, :]

    sm = jnp.mean(src_pixel_group, axis=2)
    dm = jnp.mean(dst_pixel_group, axis=2)
    d = (jnp.einsum("bik,bjck->bijc", sm, pred_R)
         + pred_T[:, None, :, 0, :] - dm[:, :, None, :])
    d = d + jnp.swapaxes(d, 1, 2)
    diff_out = jnp.transpose(d, (0, 3, 1, 2))

    U_in = jnp.concatenate(
        [diff_out.reshape(B, 2, SS),
         jnp.broadcast_to(pos_sp[:, :, :, None], (B, 2, S, S)).reshape(B, 2, SS)],
        axis=1)

    wp1, wp2, wp3 = _wT(u_pre_0_w), _wT(u_pre_1_w), _wT(u_pre_2_w)
    bp1, bp2, bp3 = _bc(u_pre_0_b), _bc(u_pre_1_b), _bc(u_pre_2_b)
    CU = wp3.shape[0]
    U, g_in = pl.pallas_call(
        functools.partial(_u_pre_kernel, s=S),
        out_shape=(jax.ShapeDtypeStruct((B, CU, SS), jnp.bfloat16),
                   jax.ShapeDtypeStruct((B, CU, 2 * S), jnp.bfloat16)),
        grid=(B,),
        in_specs=[
            pl.BlockSpec((1, 4, SS), lambda i: (i, 0, 0)),
            pl.BlockSpec(wp1.shape, lambda i: (0, 0)),
            pl.BlockSpec(bp1.shape, lambda i: (0, 0)),
            pl.BlockSpec(wp2.shape, lambda i: (0, 0)),
            pl.BlockSpec(bp2.shape, lambda i: (0, 0)),
            pl.BlockSpec(wp3.shape, lambda i: (0, 0)),
            pl.BlockSpec(bp3.shape, lambda i: (0, 0)),
        ],
        out_specs=(pl.BlockSpec((1, CU, SS), lambda i: (i, 0, 0)),
                   pl.BlockSpec((1, CU, 2 * S), lambda i: (i, 0, 0))),
        compiler_params=par,
    )(U_in, wp1, bp1, wp2, bp2, wp3, bp3)

    w1T = jnp.transpose(u_post_0_w)
    CG = u_global_2_w.shape[1]
    wu = w1T[:, :CU].astype(jnp.bfloat16)
    wg0 = w1T[:, CU:CU + CG].astype(jnp.bfloat16)
    wg1 = w1T[:, CU + CG:CU + 2 * CG].astype(jnp.bfloat16)

    wg_1, wg_2, wg_3 = _wT(u_global_0_w), _wT(u_global_1_w), _wT(u_global_2_w)
    bg_1, bg_2, bg_3 = _bc(u_global_0_b), _bc(u_global_1_b), _bc(u_global_2_b)
    pg = pl.pallas_call(
        functools.partial(_u_global_kernel, s=S),
        out_shape=jax.ShapeDtypeStruct((B, 256, 2 * S), f32),
        grid=(B,),
        in_specs=[
            pl.BlockSpec((1, CU, 2 * S), lambda i: (i, 0, 0)),
            pl.BlockSpec(wg_1.shape, lambda i: (0, 0)),
            pl.BlockSpec(bg_1.shape, lambda i: (0, 0)),
            pl.BlockSpec(wg_2.shape, lambda i: (0, 0)),
            pl.BlockSpec(bg_2.shape, lambda i: (0, 0)),
            pl.BlockSpec(wg_3.shape, lambda i: (0, 0)),
            pl.BlockSpec(bg_3.shape, lambda i: (0, 0)),
            pl.BlockSpec(wg0.shape, lambda i: (0, 0)),
            pl.BlockSpec(wg1.shape, lambda i: (0, 0)),
        ],
        out_specs=pl.BlockSpec((1, 256, 2 * S), lambda i: (i, 0, 0)),
        compiler_params=par,
    )(g_in, wg_1, bg_1, wg_2, bg_2, wg_3, bg_3, wg0, wg1)

    b1c = _bc(u_post_0_b)
    w2t, w3t, w4t = _wT(u_post_1_w), _wT(u_post_2_w), _wT(u_post_3_w)
    b2c, b3c, b4c = _bc(u_post_1_b), _bc(u_post_2_b), _bc(u_post_3_b)
    sim = pl.pallas_call(
        functools.partial(_u_post_kernel, s=S, tn=SS),
        out_shape=jax.ShapeDtypeStruct((B, 1, SS), f32),
        grid=(B, 1),
        in_specs=[
            pl.BlockSpec((1, CU, SS), lambda i, k: (i, 0, k)),
            pl.BlockSpec((1, 256, 2 * S), lambda i, k: (i, 0, 0)),
            pl.BlockSpec(wu.shape, lambda i, k: (0, 0)),
            pl.BlockSpec(b1c.shape, lambda i, k: (0, 0)),
            pl.BlockSpec(w2t.shape, lambda i, k: (0, 0)),
            pl.BlockSpec(b2c.shape, lambda i, k: (0, 0)),
            pl.BlockSpec(w3t.shape, lambda i, k: (0, 0)),
            pl.BlockSpec(b3c.shape, lambda i, k: (0, 0)),
            pl.BlockSpec(w4t.shape, lambda i, k: (0, 0)),
            pl.BlockSpec(b4c.shape, lambda i, k: (0, 0)),
        ],
        out_specs=pl.BlockSpec((1, 1, SS), lambda i, k: (i, 0, k)),
        compiler_params=par_par,
    )(U, pg, wu, b1c, w2t, b2c, w3t, b3c, w4t, b4c)
    sim = sim.reshape(B, S, S)

    seg_slic = jnp.ones((B, S, 1), f32)
    return diff_out, sim, seg_slic, pred_R, pred_T


# speedup vs baseline: 1.0134x; 1.0134x over previous
import functools

import jax
import jax.numpy as jnp
from jax import lax
from jax.experimental import pallas as pl
from jax.experimental.pallas import tpu as pltpu

_DIMS_T = (((1,), (1,)), ((), ()))


def _hilo(x):
    hi = x.astype(jnp.bfloat16)
    lo = (x - hi.astype(jnp.float32)).astype(jnp.bfloat16)
    return jnp.concatenate([hi, lo], axis=0)


def _cent_kernel(slic_ref, src_ref, tar_ref, oa_ref, ov_ref, *, s):
    k = pl.program_id(1)

    @pl.when(k == 0)
    def _():
        oa_ref[...] = jnp.zeros_like(oa_ref)
        ov_ref[...] = jnp.zeros_like(ov_ref)

    slic = slic_ref[0]
    src = src_ref[0]
    tar = tar_ref[0]
    tp = src.shape[1]
    tar_neg = (tar[0:1] < 0.0) | (tar[1:2] < 0.0)
    seg_v = jnp.where(slic < 0, s, slic)
    seg_a = jnp.where(tar_neg, s, seg_v)
    iota = lax.broadcasted_iota(jnp.int32, (s + 1, tp), 0)
    oh_a = (seg_a == iota).astype(jnp.bfloat16)
    oh_v = (seg_v == iota).astype(jnp.bfloat16)
    ones = jnp.ones((1, tp), jnp.float32)
    data_a = _hilo(jnp.concatenate([src, tar, ones], axis=0))
    data_v = _hilo(jnp.concatenate([src, ones], axis=0))
    ra = lax.dot_general(data_a, oh_a, _DIMS_T, preferred_element_type=jnp.float32)
    rv = lax.dot_general(data_v, oh_v, _DIMS_T, preferred_element_type=jnp.float32)
    oa_ref[0] += ra[:5] + ra[5:]
    ov_ref[0] += rv[:3] + rv[3:]


def _unet_tail(x, oh, wfg_ref, bfg_ref, wo_ref, bo_ref, o_ref, *, chn):
    fg = jnp.dot(wfg_ref[...], x, preferred_element_type=jnp.float32) + bfg_ref[...]
    feat = jnp.maximum(fg[:chn], 0.0)
    gate = jax.nn.sigmoid(fg[chn:])
    h = (feat * gate).astype(jnp.bfloat16)
    out = jnp.dot(wo_ref[...], h, preferred_element_type=jnp.float32) + bo_ref[...]
    out = jnp.maximum(out, 0.0)
    o_ref[0] += lax.dot_general(out.astype(jnp.bfloat16), oh,
                                _DIMS_T, preferred_element_type=jnp.float32)


def _pix_common(slic_ref, src_ref, tar_ref, *, s):
    slic = slic_ref[0]
    src = src_ref[0]
    tar = tar_ref[0]
    tp = src.shape[1]
    tar_neg = (tar[0:1] < 0.0) | (tar[1:2] < 0.0)
    seg = jnp.where(tar_neg | (slic < 0), s, slic)
    iota = lax.broadcasted_iota(jnp.int32, (s + 1, tp), 0)
    oh = (seg == iota).astype(jnp.bfloat16)
    return src, tar, tar_neg, oh


def _unet_r_kernel(slic_ref, src_ref, tar_ref, msk_ref, aux_ref,
                   wfg_ref, bfg_ref, wo_ref, bo_ref, o_ref, *, s, chn):
    k = pl.program_id(1)

    @pl.when(k == 0)
    def _():
        o_ref[...] = jnp.zeros_like(o_ref)

    src, tar, tar_neg, oh = _pix_common(slic_ref, src_ref, tar_ref, s=s)
    g2 = jnp.dot(_hilo(aux_ref[0]), oh, preferred_element_type=jnp.float32)
    g = g2[:4] + g2[4:]
    pm = jnp.concatenate([src - g[:2], tar - g[2:4]], axis=0)
    pm = jnp.where(jnp.logical_not(tar_neg), pm, -1.0)
    x = jnp.concatenate([pm, msk_ref[0]], axis=0).astype(jnp.bfloat16)
    _unet_tail(x, oh, wfg_ref, bfg_ref, wo_ref, bo_ref, o_ref, chn=chn)


def _unet_t_kernel(slic_ref, src_ref, tar_ref, msk_ref, sumr_ref, cnt_ref,
                   mw0_ref, mb0_ref, mw1_ref, mb1_ref,
                   wfg_ref, bfg_ref, wo_ref, bo_ref,
                   o_ref, pred_ref, pr_scr, *, s, chn):
    k = pl.program_id(1)

    @pl.when(k == 0)
    def _():
        o_ref[...] = jnp.zeros_like(o_ref)
        feat = sumr_ref[0] / jnp.maximum(cnt_ref[0], 1.0)
        hh = jnp.maximum(
            jnp.dot(mw0_ref[...], feat.astype(jnp.bfloat16),
                    preferred_element_type=jnp.float32) + mb0_ref[...], 0.0)
        pred = jnp.dot(mw1_ref[...], hh.astype(jnp.bfloat16),
                       preferred_element_type=jnp.float32) + mb1_ref[...]
        pr_scr[...] = pred
        pred_ref[0] = pred

    src, tar, tar_neg, oh = _pix_common(slic_ref, src_ref, tar_ref, s=s)
    g2 = jnp.dot(_hilo(pr_scr[...]), oh, preferred_element_type=jnp.float32)
    g = g2[:2] + g2[2:]
    a = g[0:1]
    b = g[1:2]
    rx = src[0:1] * (1.0 + a) + src[1:2] * b
    ry = -src[0:1] * b + src[1:2] * (1.0 + a)
    pm = jnp.concatenate([rx, ry, tar], axis=0)
    pm = jnp.where(jnp.logical_not(tar_neg), pm, -1.0)
    x = jnp.concatenate([pm, msk_ref[0]], axis=0).astype(jnp.bfloat16)
    _unet_tail(x, oh, wfg_ref, bfg_ref, wo_ref, bo_ref, o_ref, chn=chn)


def _mreg_kernel(s_ref, c_ref, w0_ref, b0_ref, w1_ref, b1_ref, o_ref):
    feat = s_ref[0] / jnp.maximum(c_ref[0], 1.0)
    h = jnp.dot(w0_ref[...], feat.astype(jnp.bfloat16),
                preferred_element_type=jnp.float32) + b0_ref[...]
    h = jnp.maximum(h, 0.0)
    o_ref[0] = jnp.dot(w1_ref[...], h.astype(jnp.bfloat16),
                       preferred_element_type=jnp.float32) + b1_ref[...]


def _u_pre_kernel(as_ref, al_ref, w1_ref, b1_ref, w2_ref, b2_ref, w3_ref, b3_ref,
                  u_ref, g_ref, d_ref, *, s):
    A = as_ref[0]
    L = al_ref[0]
    D0 = (A[:, 0:1] * L[4:5] + A[:, 1:2] * L[5:6] + L[8:9] - A[:, 2:3]
          + A[:, 4:5] * L[0:1] + A[:, 5:6] * L[1:2] + A[:, 8:9] - L[2:3])
    D1 = (A[:, 0:1] * L[6:7] + A[:, 1:2] * L[7:8] + L[9:10] - A[:, 3:4]
          + A[:, 6:7] * L[0:1] + A[:, 7:8] * L[1:2] + A[:, 9:10] - L[3:4])
    P0 = jnp.broadcast_to(A[:, 10:11], (s, s))
    P1 = jnp.broadcast_to(A[:, 11:12], (s, s))
    d_ref[0] = jnp.stack([D0, D1], axis=0)
    x = jnp.stack([D0, D1, P0, P1], axis=0).reshape(4, s * s).astype(jnp.bfloat16)

    h = jnp.maximum(jnp.dot(w1_ref[...], x,
                            preferred_element_type=jnp.float32) + b1_ref[...], 0.0)
    h = jnp.maximum(jnp.dot(w2_ref[...], h.astype(jnp.bfloat16),
                            preferred_element_type=jnp.float32) + b2_ref[...], 0.0)
    h = jnp.maximum(jnp.dot(w3_ref[...], h.astype(jnp.bfloat16),
                            preferred_element_type=jnp.float32) + b3_ref[...], 0.0)
    u = h.astype(jnp.bfloat16)
    u_ref[0] = u
    u3 = u.reshape(u.shape[0], s, s)
    g_ref[0] = jnp.concatenate([jnp.max(u3, axis=2), jnp.max(u3, axis=1)], axis=1)


def _u_post_kernel(u_ref, gin_ref, g1w_ref, g1b_ref, g2w_ref, g2b_ref,
                   g3w_ref, g3b_ref, wg0_ref, wg1_ref,
                   wu_ref, b1_ref, w2_ref, b2_ref, w3_ref, b3_ref,
                   w4_ref, b4_ref, o_ref, *, s):
    xg = gin_ref[0]
    hg = jnp.maximum(jnp.dot(g1w_ref[...], xg,
                             preferred_element_type=jnp.float32) + g1b_ref[...], 0.0)
    hg = jnp.maximum(jnp.dot(g2w_ref[...], hg.astype(jnp.bfloat16),
                             preferred_element_type=jnp.float32) + g2b_ref[...], 0.0)
    hg = jnp.maximum(jnp.dot(g3w_ref[...], hg.astype(jnp.bfloat16),
                             preferred_element_type=jnp.float32) + g3b_ref[...], 0.0)
    g = hg.astype(jnp.bfloat16)
    pg = jnp.concatenate(
        [jnp.dot(wg0_ref[...], g[:, :s], preferred_element_type=jnp.float32),
         jnp.dot(wg1_ref[...], g[:, s:], preferred_element_type=jnp.float32)],
        axis=1).astype(jnp.bfloat16)

    n = u_ref.shape[2]
    p = lax.broadcasted_iota(jnp.int32, (1, n), 1)
    rid = p // s
    cid = p - rid * s
    riota = lax.broadcasted_iota(jnp.int32, (s, n), 0)
    sel = jnp.concatenate([(rid == riota).astype(jnp.bfloat16),
                           (cid == riota).astype(jnp.bfloat16)], axis=0)
    h = jnp.dot(wu_ref[...], u_ref[0], preferred_element_type=jnp.float32)
    h = h + jnp.dot(pg, sel, preferred_element_type=jnp.float32)
    h = jnp.maximum(h + b1_ref[...], 0.0)
    h = jnp.maximum(jnp.dot(w2_ref[...], h.astype(jnp.bfloat16),
                            preferred_element_type=jnp.float32) + b2_ref[...], 0.0)
    h = jnp.maximum(jnp.dot(w3_ref[...], h.astype(jnp.bfloat16),
                            preferred_element_type=jnp.float32) + b3_ref[...], 0.0)
    o_ref[0] = jnp.dot(w4_ref[...], h.astype(jnp.bfloat16),
                       preferred_element_type=jnp.float32) + b4_ref[...]


def _wT(w):
    return jnp.transpose(w).astype(jnp.bfloat16)


def _bc(b):
    return b.reshape(-1, 1).astype(jnp.float32)


def kernel(pos_src, pos_tar, mask, slic_map, src_pixel_group, dst_pixel_group,
           unet_r_feat_w, unet_r_feat_b, unet_r_gate_w, unet_r_gate_b,
           unet_r_out_w, unet_r_out_b,
           unet_t_feat_w, unet_t_feat_b, unet_t_gate_w, unet_t_gate_b,
           unet_t_out_w, unet_t_out_b,
           mreg_r_0_w, mreg_r_0_b, mreg_r_1_w, mreg_r_1_b,
           mreg_t_0_w, mreg_t_0_b, mreg_t_1_w, mreg_t_1_b,
           u_pre_0_w, u_pre_0_b, u_pre_1_w, u_pre_1_b, u_pre_2_w, u_pre_2_b,
           u_global_0_w, u_global_0_b, u_global_1_w, u_global_1_b,
           u_global_2_w, u_global_2_b,
           u_post_0_w, u_post_0_b, u_post_1_w, u_post_1_b,
           u_post_2_w, u_post_2_b, u_post_3_w, u_post_3_b):
    B, _, H, W = pos_src.shape
    P = H * W
    S = src_pixel_group.shape[1]
    S1 = S + 1
    SS = S * S
    f32 = jnp.float32

    src = pos_src.reshape(B, 2, P)
    tar = pos_tar.reshape(B, 2, P)
    msk = mask.reshape(B, 1, P)
    slic = slic_map.reshape(B, 1, P).astype(jnp.int32)

    nk = 2 if P % 2 == 0 else 1
    TP = P // nk
    par_arb = pltpu.CompilerParams(dimension_semantics=("parallel", "arbitrary"))
    par = pltpu.CompilerParams(dimension_semantics=("parallel",))

    sums_a, sums_v = pl.pallas_call(
        functools.partial(_cent_kernel, s=S),
        out_shape=(jax.ShapeDtypeStruct((B, 5, S1), f32),
                   jax.ShapeDtypeStruct((B, 3, S1), f32)),
        grid=(B, nk),
        in_specs=[
            pl.BlockSpec((1, 1, TP), lambda i, k: (i, 0, k)),
            pl.BlockSpec((1, 2, TP), lambda i, k: (i, 0, k)),
            pl.BlockSpec((1, 2, TP), lambda i, k: (i, 0, k)),
        ],
        out_specs=(pl.BlockSpec((1, 5, S1), lambda i, k: (i, 0, 0)),
                   pl.BlockSpec((1, 3, S1), lambda i, k: (i, 0, 0))),
        compiler_params=par_arb,
    )(slic, src, tar)

    counts = sums_a[:, 4:5]
    cent = sums_a[:, :4] / jnp.maximum(counts, 1.0)
    pos_sp = (sums_v[:, :2] / jnp.maximum(sums_v[:, 2:3], 1.0))[:, :, :S]

    def unet_w(fw, fb, gw, gb, ow, ob):
        wfg = jnp.transpose(jnp.concatenate([fw, gw], axis=1)).astype(jnp.bfloat16)
        bfg = jnp.concatenate([fb, gb]).reshape(-1, 1).astype(f32)
        return wfg, bfg, _wT(ow), _bc(ob)

    pix_specs = [
        pl.BlockSpec((1, 1, TP), lambda i, k: (i, 0, k)),
        pl.BlockSpec((1, 2, TP), lambda i, k: (i, 0, k)),
        pl.BlockSpec((1, 2, TP), lambda i, k: (i, 0, k)),
        pl.BlockSpec((1, 1, TP), lambda i, k: (i, 0, k)),
    ]

    def cspec(shape):
        if len(shape) == 3:
            return pl.BlockSpec(shape, lambda i, k: (i, 0, 0))
        return pl.BlockSpec(shape, lambda i, k: (0, 0))

    chn = unet_r_feat_w.shape[1]
    wfg, bfg, wo, bo = unet_w(unet_r_feat_w, unet_r_feat_b, unet_r_gate_w,
                              unet_r_gate_b, unet_r_out_w, unet_r_out_b)
    sum_R = pl.pallas_call(
        functools.partial(_unet_r_kernel, s=S, chn=chn),
        out_shape=jax.ShapeDtypeStruct((B, chn, S1), f32),
        grid=(B, nk),
        in_specs=pix_specs + [cspec((1, 4, S1)), cspec(wfg.shape), cspec(bfg.shape),
                              cspec(wo.shape), cspec(bo.shape)],
        out_specs=pl.BlockSpec((1, chn, S1), lambda i, k: (i, 0, 0)),
        compiler_params=par_arb,
    )(slic, src, tar, msk, cent, wfg, bfg, wo, bo)

    mw0, mw1 = _wT(mreg_r_0_w), _wT(mreg_r_1_w)
    mb0, mb1 = _bc(mreg_r_0_b), _bc(mreg_r_1_b)
    wfg_t, bfg_t, wo_t, bo_t = unet_w(unet_t_feat_w, unet_t_feat_b, unet_t_gate_w,
                                      unet_t_gate_b, unet_t_out_w, unet_t_out_b)
    sum_T, pred_ab = pl.pallas_call(
        functools.partial(_unet_t_kernel, s=S, chn=chn),
        out_shape=(jax.ShapeDtypeStruct((B, chn, S1), f32),
                   jax.ShapeDtypeStruct((B, 2, S1), f32)),
        grid=(B, nk),
        in_specs=pix_specs + [cspec((1, chn, S1)), cspec((1, 1, S1)),
                              cspec(mw0.shape), cspec(mb0.shape),
                              cspec(mw1.shape), cspec(mb1.shape),
                              cspec(wfg_t.shape), cspec(bfg_t.shape),
                              cspec(wo_t.shape), cspec(bo_t.shape)],
        out_specs=(pl.BlockSpec((1, chn, S1), lambda i, k: (i, 0, 0)),
                   pl.BlockSpec((1, 2, S1), lambda i, k: (i, 0, 0))),
        scratch_shapes=[pltpu.VMEM((2, S1), f32)],
        compiler_params=par_arb,
    )(slic, src, tar, msk, sum_R, counts, mw0, mb0, mw1, mb1,
      wfg_t, bfg_t, wo_t, bo_t)

    a = pred_ab[:, 0, :S]
    b = pred_ab[:, 1, :S]
    pred_R = jnp.stack([jnp.stack([1.0 + a, -b], axis=-1),
                        jnp.stack([b, 1.0 + a], axis=-1)], axis=-2)

    tw0, tw1 = _wT(mreg_t_0_w), _wT(mreg_t_1_w)
    tb0, tb1 = _bc(mreg_t_0_b), _bc(mreg_t_1_b)
    pred_t_ab = pl.pallas_call(
        _mreg_kernel,
        out_shape=jax.ShapeDtypeStruct((B, 2, S1), f32),
        grid=(B,),
        in_specs=[
            pl.BlockSpec((1, chn, S1), lambda i: (i, 0, 0)),
            pl.BlockSpec((1, 1, S1), lambda i: (i, 0, 0)),
            pl.BlockSpec(tw0.shape, lambda i: (0, 0)),
            pl.BlockSpec(tb0.shape, lambda i: (0, 0)),
            pl.BlockSpec(tw1.shape, lambda i: (0, 0)),
            pl.BlockSpec(tb1.shape, lambda i: (0, 0)),
        ],
        out_specs=pl.BlockSpec((1, 2, S1), lambda i: (i, 0, 0)),
        compiler_params=par,
    )(sum_T, counts, tw0, tb0, tw1, tb1)
    pred_T = jnp.transpose(pred_t_ab, (0, 2, 1))[:, :S][:, :, None, :]

    sm = jnp.mean(src_pixel_group, axis=2)
    dm = jnp.mean(dst_pixel_group, axis=2)
    t0 = pred_t_ab[:, 0, :S]
    t1 = pred_t_ab[:, 1, :S]
    AS = jnp.stack([sm[..., 0], sm[..., 1], dm[..., 0], dm[..., 1],
                    1.0 + a, -b, b, 1.0 + a, t0, t1,
                    pos_sp[:, 0], pos_sp[:, 1]], axis=-1)
    AL = jnp.swapaxes(AS, 1, 2)

    wp1, wp2, wp3 = _wT(u_pre_0_w), _wT(u_pre_1_w), _wT(u_pre_2_w)
    bp1, bp2, bp3 = _bc(u_pre_0_b), _bc(u_pre_1_b), _bc(u_pre_2_b)
    CU = wp3.shape[0]
    U, g_in, diff_out = pl.pallas_call(
        functools.partial(_u_pre_kernel, s=S),
        out_shape=(jax.ShapeDtypeStruct((B, CU, SS), jnp.bfloat16),
                   jax.ShapeDtypeStruct((B, CU, 2 * S), jnp.bfloat16),
                   jax.ShapeDtypeStruct((B, 2, S, S), f32)),
        grid=(B,),
        in_specs=[
            pl.BlockSpec((1, S, 12), lambda i: (i, 0, 0)),
            pl.BlockSpec((1, 12, S), lambda i: (i, 0, 0)),
            pl.BlockSpec(wp1.shape, lambda i: (0, 0)),
            pl.BlockSpec(bp1.shape, lambda i: (0, 0)),
            pl.BlockSpec(wp2.shape, lambda i: (0, 0)),
            pl.BlockSpec(bp2.shape, lambda i: (0, 0)),
            pl.BlockSpec(wp3.shape, lambda i: (0, 0)),
            pl.BlockSpec(bp3.shape, lambda i: (0, 0)),
        ],
        out_specs=(pl.BlockSpec((1, CU, SS), lambda i: (i, 0, 0)),
                   pl.BlockSpec((1, CU, 2 * S), lambda i: (i, 0, 0)),
                   pl.BlockSpec((1, 2, S, S), lambda i: (i, 0, 0, 0))),
        compiler_params=par,
    )(AS, AL, wp1, bp1, wp2, bp2, wp3, bp3)

    w1T = jnp.transpose(u_post_0_w)
    CG = u_global_2_w.shape[1]
    wu = w1T[:, :CU].astype(jnp.bfloat16)
    wg0 = w1T[:, CU:CU + CG].astype(jnp.bfloat16)
    wg1 = w1T[:, CU + CG:CU + 2 * CG].astype(jnp.bfloat16)
    wg_1, wg_2, wg_3 = _wT(u_global_0_w), _wT(u_global_1_w), _wT(u_global_2_w)
    bg_1, bg_2, bg_3 = _bc(u_global_0_b), _bc(u_global_1_b), _bc(u_global_2_b)
    b1c = _bc(u_post_0_b)
    w2t, w3t, w4t = _wT(u_post_1_w), _wT(u_post_2_w), _wT(u_post_3_w)
    b2c, b3c, b4c = _bc(u_post_1_b), _bc(u_post_2_b), _bc(u_post_3_b)

    def ws(shape):
        return pl.BlockSpec(shape, lambda i: (0, 0))

    sim = pl.pallas_call(
        functools.partial(_u_post_kernel, s=S),
        out_shape=jax.ShapeDtypeStruct((B, 1, SS), f32),
        grid=(B,),
        in_specs=[
            pl.BlockSpec((1, CU, SS), lambda i: (i, 0, 0)),
            pl.BlockSpec((1, CU, 2 * S), lambda i: (i, 0, 0)),
            ws(wg_1.shape), ws(bg_1.shape), ws(wg_2.shape), ws(bg_2.shape),
            ws(wg_3.shape), ws(bg_3.shape), ws(wg0.shape), ws(wg1.shape),
            ws(wu.shape), ws(b1c.shape), ws(w2t.shape), ws(b2c.shape),
            ws(w3t.shape), ws(b3c.shape), ws(w4t.shape), ws(b4c.shape),
        ],
        out_specs=pl.BlockSpec((1, 1, SS), lambda i: (i, 0, 0)),
        compiler_params=par,
    )(U, g_in, wg_1, bg_1, wg_2, bg_2, wg_3, bg_3, wg0, wg1,
      wu, b1c, w2t, b2c, w3t, b3c, w4t, b4c)
    sim = sim.reshape(B, S, S)

    seg_slic = jnp.ones((B, S, 1), f32)
    return diff_out, sim, seg_slic, pred_R, pred_T


# speedup vs baseline: 1.0141x; 1.0008x over previous
import functools

import jax
import jax.numpy as jnp
from jax import lax
from jax.experimental import pallas as pl
from jax.experimental.pallas import tpu as pltpu

_DIMS_T = (((1,), (1,)), ((), ()))


def _hilo(x):
    hi = x.astype(jnp.bfloat16)
    lo = (x - hi.astype(jnp.float32)).astype(jnp.bfloat16)
    return jnp.concatenate([hi, lo], axis=0)


def _mreg(feat, w0_ref, b0_ref, w1_ref, b1_ref):
    h = jnp.dot(w0_ref[...], feat.astype(jnp.bfloat16),
                preferred_element_type=jnp.float32) + b0_ref[...]
    h = jnp.maximum(h, 0.0)
    return jnp.dot(w1_ref[...], h.astype(jnp.bfloat16),
                   preferred_element_type=jnp.float32) + b1_ref[...]


def _unet(x, oh, wfg_ref, bfg_ref, wo_ref, bo_ref, *, chn):
    fg = jnp.dot(wfg_ref[...], x, preferred_element_type=jnp.float32) + bfg_ref[...]
    feat = jnp.maximum(fg[:chn], 0.0)
    gate = jax.nn.sigmoid(fg[chn:])
    h = (feat * gate).astype(jnp.bfloat16)
    out = jnp.dot(wo_ref[...], h, preferred_element_type=jnp.float32) + bo_ref[...]
    out = jnp.maximum(out, 0.0)
    return lax.dot_general(out.astype(jnp.bfloat16), oh, _DIMS_T,
                           preferred_element_type=jnp.float32)


def _pix_kernel(slic_ref, src_ref, tar_ref, msk_ref,
                mrw0_ref, mrb0_ref, mrw1_ref, mrb1_ref,
                mtw0_ref, mtb0_ref, mtw1_ref, mtb1_ref,
                wfgr_ref, bfgr_ref, wor_ref, bor_ref,
                wfgt_ref, bfgt_ref, wot_ref, bot_ref,
                ps_ref, pra_ref, prt_ref,
                oh_scr, cent_scr, cnt_scr, pred_scr, *, s, chn):
    k = pl.program_id(1)
    src = src_ref[0]
    tar = tar_ref[0]
    tar_neg = (tar[0:1] < 0.0) | (tar[1:2] < 0.0)

    @pl.when(k == 0)
    def _():
        slic = slic_ref[0]
        p_n = src.shape[1]
        seg_v = jnp.where(slic < 0, s, slic)
        seg_a = jnp.where(tar_neg, s, seg_v)
        iota = lax.broadcasted_iota(jnp.int32, (s + 1, p_n), 0)
        oh_a = (seg_a == iota).astype(jnp.bfloat16)
        oh_v = (seg_v == iota).astype(jnp.bfloat16)
        oh_scr[...] = oh_a
        ones = jnp.ones((1, p_n), jnp.float32)
        da = _hilo(jnp.concatenate([src, tar, ones], axis=0))
        dv = _hilo(jnp.concatenate([src, ones], axis=0))
        ra = lax.dot_general(da, oh_a, _DIMS_T, preferred_element_type=jnp.float32)
        rv = lax.dot_general(dv, oh_v, _DIMS_T, preferred_element_type=jnp.float32)
        sums_a = ra[:5] + ra[5:]
        sums_v = rv[:3] + rv[3:]
        cnt = sums_a[4:5]
        cnt_scr[...] = cnt
        cent_scr[...] = sums_a[:4] / jnp.maximum(cnt, 1.0)
        ps_ref[0] = (sums_v[:2] / jnp.maximum(sums_v[2:3], 1.0))[:, :s]

    @pl.when(k == 1)
    def _():
        oh = oh_scr[...]
        g2 = jnp.dot(_hilo(cent_scr[...]), oh, preferred_element_type=jnp.float32)
        g = g2[:4] + g2[4:]
        pm = jnp.concatenate([src - g[:2], tar - g[2:4]], axis=0)
        pm = jnp.where(jnp.logical_not(tar_neg), pm, -1.0)
        x = jnp.concatenate([pm, msk_ref[0]], axis=0).astype(jnp.bfloat16)
        sum_r = _unet(x, oh, wfgr_ref, bfgr_ref, wor_ref, bor_ref, chn=chn)
        feat = sum_r / jnp.maximum(cnt_scr[...], 1.0)
        pred = _mreg(feat, mrw0_ref, mrb0_ref, mrw1_ref, mrb1_ref)
        pred_scr[...] = pred
        pra_ref[0] = pred

    @pl.when(k == 2)
    def _():
        oh = oh_scr[...]
        g2 = jnp.dot(_hilo(pred_scr[...]), oh, preferred_element_type=jnp.float32)
        g = g2[:2] + g2[2:]
        a = g[0:1]
        b = g[1:2]
        rx = src[0:1] * (1.0 + a) + src[1:2] * b
        ry = -src[0:1] * b + src[1:2] * (1.0 + a)
        pm = jnp.concatenate([rx, ry, tar], axis=0)
        pm = jnp.where(jnp.logical_not(tar_neg), pm, -1.0)
        x = jnp.concatenate([pm, msk_ref[0]], axis=0).astype(jnp.bfloat16)
        sum_t = _unet(x, oh, wfgt_ref, bfgt_ref, wot_ref, bot_ref, chn=chn)
        feat = sum_t / jnp.maximum(cnt_scr[...], 1.0)
        prt_ref[0] = _mreg(feat, mtw0_ref, mtb0_ref, mtw1_ref, mtb1_ref)


def _u_pre_kernel(as_ref, al_ref, w1_ref, b1_ref, w2_ref, b2_ref, w3_ref, b3_ref,
                  u_ref, d_ref, *, s):
    A = as_ref[0]
    L = al_ref[0]
    D0 = (A[:, 0:1] * L[4:5] + A[:, 1:2] * L[5:6] + L[8:9] - A[:, 2:3]
          + A[:, 4:5] * L[0:1] + A[:, 5:6] * L[1:2] + A[:, 8:9] - L[2:3])
    D1 = (A[:, 0:1] * L[6:7] + A[:, 1:2] * L[7:8] + L[9:10] - A[:, 3:4]
          + A[:, 6:7] * L[0:1] + A[:, 7:8] * L[1:2] + A[:, 9:10] - L[3:4])
    P0 = jnp.broadcast_to(A[:, 10:11], (s, s))
    P1 = jnp.broadcast_to(A[:, 11:12], (s, s))
    d_ref[0] = jnp.stack([D0, D1], axis=0)
    x = jnp.stack([D0, D1, P0, P1], axis=0).reshape(4, s * s).astype(jnp.bfloat16)

    h = jnp.maximum(jnp.dot(w1_ref[...], x,
                            preferred_element_type=jnp.float32) + b1_ref[...], 0.0)
    h = jnp.maximum(jnp.dot(w2_ref[...], h.astype(jnp.bfloat16),
                            preferred_element_type=jnp.float32) + b2_ref[...], 0.0)
    h = jnp.maximum(jnp.dot(w3_ref[...], h.astype(jnp.bfloat16),
                            preferred_element_type=jnp.float32) + b3_ref[...], 0.0)
    u_ref[0] = h.astype(jnp.bfloat16)


def _u_post_kernel(u_ref, gin_ref, g1w_ref, g1b_ref, g2w_ref, g2b_ref,
                   g3w_ref, g3b_ref, wg0_ref, wg1_ref,
                   wu_ref, b1_ref, w2_ref, b2_ref, w3_ref, b3_ref,
                   w4_ref, b4_ref, o_ref, *, s):
    xg = gin_ref[0]
    hg = jnp.maximum(jnp.dot(g1w_ref[...], xg,
                             preferred_element_type=jnp.float32) + g1b_ref[...], 0.0)
    hg = jnp.maximum(jnp.dot(g2w_ref[...], hg.astype(jnp.bfloat16),
                             preferred_element_type=jnp.float32) + g2b_ref[...], 0.0)
    hg = jnp.maximum(jnp.dot(g3w_ref[...], hg.astype(jnp.bfloat16),
                             preferred_element_type=jnp.float32) + g3b_ref[...], 0.0)
    g = hg.astype(jnp.bfloat16)
    pg = jnp.concatenate(
        [jnp.dot(wg0_ref[...], g[:, :s], preferred_element_type=jnp.float32),
         jnp.dot(wg1_ref[...], g[:, s:], preferred_element_type=jnp.float32)],
        axis=1).astype(jnp.bfloat16)

    n = u_ref.shape[2]
    p = lax.broadcasted_iota(jnp.int32, (1, n), 1)
    rid = p // s
    cid = p - rid * s
    riota = lax.broadcasted_iota(jnp.int32, (s, n), 0)
    sel = jnp.concatenate([(rid == riota).astype(jnp.bfloat16),
                           (cid == riota).astype(jnp.bfloat16)], axis=0)
    h = jnp.dot(wu_ref[...], u_ref[0], preferred_element_type=jnp.float32)
    h = h + jnp.dot(pg, sel, preferred_element_type=jnp.float32)
    h = jnp.maximum(h + b1_ref[...], 0.0)
    h = jnp.maximum(jnp.dot(w2_ref[...], h.astype(jnp.bfloat16),
                            preferred_element_type=jnp.float32) + b2_ref[...], 0.0)
    h = jnp.maximum(jnp.dot(w3_ref[...], h.astype(jnp.bfloat16),
                            preferred_element_type=jnp.float32) + b3_ref[...], 0.0)
    o_ref[0] = jnp.dot(w4_ref[...], h.astype(jnp.bfloat16),
                       preferred_element_type=jnp.float32) + b4_ref[...]


def _wT(w):
    return jnp.transpose(w).astype(jnp.bfloat16)


def _bc(b):
    return b.reshape(-1, 1).astype(jnp.float32)


def kernel(pos_src, pos_tar, mask, slic_map, src_pixel_group, dst_pixel_group,
           unet_r_feat_w, unet_r_feat_b, unet_r_gate_w, unet_r_gate_b,
           unet_r_out_w, unet_r_out_b,
           unet_t_feat_w, unet_t_feat_b, unet_t_gate_w, unet_t_gate_b,
           unet_t_out_w, unet_t_out_b,
           mreg_r_0_w, mreg_r_0_b, mreg_r_1_w, mreg_r_1_b,
           mreg_t_0_w, mreg_t_0_b, mreg_t_1_w, mreg_t_1_b,
           u_pre_0_w, u_pre_0_b, u_pre_1_w, u_pre_1_b, u_pre_2_w, u_pre_2_b,
           u_global_0_w, u_global_0_b, u_global_1_w, u_global_1_b,
           u_global_2_w, u_global_2_b,
           u_post_0_w, u_post_0_b, u_post_1_w, u_post_1_b,
           u_post_2_w, u_post_2_b, u_post_3_w, u_post_3_b):
    B, _, H, W = pos_src.shape
    P = H * W
    S = src_pixel_group.shape[1]
    S1 = S + 1
    SS = S * S
    f32 = jnp.float32

    src = pos_src.reshape(B, 2, P)
    tar = pos_tar.reshape(B, 2, P)
    msk = mask.reshape(B, 1, P)
    slic = slic_map.reshape(B, 1, P).astype(jnp.int32)

    par_arb = pltpu.CompilerParams(dimension_semantics=("parallel", "arbitrary"))
    par = pltpu.CompilerParams(dimension_semantics=("parallel",))

    def unet_w(fw, fb, gw, gb, ow, ob):
        wfg = jnp.transpose(jnp.concatenate([fw, gw], axis=1)).astype(jnp.bfloat16)
        bfg = jnp.concatenate([fb, gb]).reshape(-1, 1).astype(f32)
        return wfg, bfg, _wT(ow), _bc(ob)

    chn = unet_r_feat_w.shape[1]
    wfgr, bfgr, wor, bor = unet_w(unet_r_feat_w, unet_r_feat_b, unet_r_gate_w,
                                  unet_r_gate_b, unet_r_out_w, unet_r_out_b)
    wfgt, bfgt, wot, bot = unet_w(unet_t_feat_w, unet_t_feat_b, unet_t_gate_w,
                                  unet_t_gate_b, unet_t_out_w, unet_t_out_b)
    mrw0, mrw1 = _wT(mreg_r_0_w), _wT(mreg_r_1_w)
    mrb0, mrb1 = _bc(mreg_r_0_b), _bc(mreg_r_1_b)
    mtw0, mtw1 = _wT(mreg_t_0_w), _wT(mreg_t_1_w)
    mtb0, mtb1 = _bc(mreg_t_0_b), _bc(mreg_t_1_b)

    def pspec(c):
        return pl.BlockSpec((1, c, P), lambda i, k: (i, 0, 0))

    def wspec(shape):
        return pl.BlockSpec(shape, lambda i, k: (0, 0))

    pos_sp, pred_ab, pred_t_ab = pl.pallas_call(
        functools.partial(_pix_kernel, s=S, chn=chn),
        out_shape=(jax.ShapeDtypeStruct((B, 2, S), f32),
                   jax.ShapeDtypeStruct((B, 2, S1), f32),
                   jax.ShapeDtypeStruct((B, 2, S1), f32)),
        grid=(B, 3),
        in_specs=[
            pspec(1), pspec(2), pspec(2), pspec(1),
            wspec(mrw0.shape), wspec(mrb0.shape), wspec(mrw1.shape), wspec(mrb1.shape),
            wspec(mtw0.shape), wspec(mtb0.shape), wspec(mtw1.shape), wspec(mtb1.shape),
            wspec(wfgr.shape), wspec(bfgr.shape), wspec(wor.shape), wspec(bor.shape),
            wspec(wfgt.shape), wspec(bfgt.shape), wspec(wot.shape), wspec(bot.shape),
        ],
        out_specs=(pl.BlockSpec((1, 2, S), lambda i, k: (i, 0, 0)),
                   pl.BlockSpec((1, 2, S1), lambda i, k: (i, 0, 0)),
                   pl.BlockSpec((1, 2, S1), lambda i, k: (i, 0, 0))),
        scratch_shapes=[
            pltpu.VMEM((S1, P), jnp.bfloat16),
            pltpu.VMEM((4, S1), f32),
            pltpu.VMEM((1, S1), f32),
            pltpu.VMEM((2, S1), f32),
        ],
        compiler_params=par_arb,
    )(slic, src, tar, msk,
      mrw0, mrb0, mrw1, mrb1, mtw0, mtb0, mtw1, mtb1,
      wfgr, bfgr, wor, bor, wfgt, bfgt, wot, bot)

    a = pred_ab[:, 0, :S]
    b = pred_ab[:, 1, :S]
    pred_R = jnp.stack([jnp.stack([1.0 + a, -b], axis=-1),
                        jnp.stack([b, 1.0 + a], axis=-1)], axis=-2)
    pred_T = jnp.transpose(pred_t_ab, (0, 2, 1))[:, :S][:, :, None, :]

    sm = jnp.mean(src_pixel_group, axis=2)
    dm = jnp.mean(dst_pixel_group, axis=2)
    t0 = pred_t_ab[:, 0, :S]
    t1 = pred_t_ab[:, 1, :S]
    AS = jnp.stack([sm[..., 0], sm[..., 1], dm[..., 0], dm[..., 1],
                    1.0 + a, -b, b, 1.0 + a, t0, t1,
                    pos_sp[:, 0], pos_sp[:, 1]], axis=-1)
    AL = jnp.swapaxes(AS, 1, 2)

    wp1, wp2, wp3 = _wT(u_pre_0_w), _wT(u_pre_1_w), _wT(u_pre_2_w)
    bp1, bp2, bp3 = _bc(u_pre_0_b), _bc(u_pre_1_b), _bc(u_pre_2_b)
    CU = wp3.shape[0]
    U, diff_out = pl.pallas_call(
        functools.partial(_u_pre_kernel, s=S),
        out_shape=(jax.ShapeDtypeStruct((B, CU, SS), jnp.bfloat16),
                   jax.ShapeDtypeStruct((B, 2, S, S), f32)),
        grid=(B,),
        in_specs=[
            pl.BlockSpec((1, S, 12), lambda i: (i, 0, 0)),
            pl.BlockSpec((1, 12, S), lambda i: (i, 0, 0)),
            pl.BlockSpec(wp1.shape, lambda i: (0, 0)),
            pl.BlockSpec(bp1.shape, lambda i: (0, 0)),
            pl.BlockSpec(wp2.shape, lambda i: (0, 0)),
            pl.BlockSpec(bp2.shape, lambda i: (0, 0)),
            pl.BlockSpec(wp3.shape, lambda i: (0, 0)),
            pl.BlockSpec(bp3.shape, lambda i: (0, 0)),
        ],
        out_specs=(pl.BlockSpec((1, CU, SS), lambda i: (i, 0, 0)),
                   pl.BlockSpec((1, 2, S, S), lambda i: (i, 0, 0, 0))),
        compiler_params=par,
    )(AS, AL, wp1, bp1, wp2, bp2, wp3, bp3)

    U4 = U.reshape(B, CU, S, S)
    g_in = jnp.concatenate([jnp.max(U4, axis=3), jnp.max(U4, axis=2)], axis=-1)

    w1T = jnp.transpose(u_post_0_w)
    CG = u_global_2_w.shape[1]
    wu = w1T[:, :CU].astype(jnp.bfloat16)
    wg0 = w1T[:, CU:CU + CG].astype(jnp.bfloat16)
    wg1 = w1T[:, CU + CG:CU + 2 * CG].astype(jnp.bfloat16)
    wg_1, wg_2, wg_3 = _wT(u_global_0_w), _wT(u_global_1_w), _wT(u_global_2_w)
    bg_1, bg_2, bg_3 = _bc(u_global_0_b), _bc(u_global_1_b), _bc(u_global_2_b)
    b1c = _bc(u_post_0_b)
    w2t, w3t, w4t = _wT(u_post_1_w), _wT(u_post_2_w), _wT(u_post_3_w)
    b2c, b3c, b4c = _bc(u_post_1_b), _bc(u_post_2_b), _bc(u_post_3_b)

    def ws(shape):
        return pl.BlockSpec(shape, lambda i: (0, 0))

    sim = pl.pallas_call(
        functools.partial(_u_post_kernel, s=S),
        out_shape=jax.ShapeDtypeStruct((B, 1, SS), f32),
        grid=(B,),
        in_specs=[
            pl.BlockSpec((1, CU, SS), lambda i: (i, 0, 0)),
            pl.BlockSpec((1, CU, 2 * S), lambda i: (i, 0, 0)),
            ws(wg_1.shape), ws(bg_1.shape), ws(wg_2.shape), ws(bg_2.shape),
            ws(wg_3.shape), ws(bg_3.shape), ws(wg0.shape), ws(wg1.shape),
            ws(wu.shape), ws(b1c.shape), ws(w2t.shape), ws(b2c.shape),
            ws(w3t.shape), ws(b3c.shape), ws(w4t.shape), ws(b4c.shape),
        ],
        out_specs=pl.BlockSpec((1, 1, SS), lambda i: (i, 0, 0)),
        compiler_params=par,
    )(U, g_in, wg_1, bg_1, wg_2, bg_2, wg_3, bg_3, wg0, wg1,
      wu, b1c, w2t, b2c, w3t, b3c, w4t, b4c)
    sim = sim.reshape(B, S, S)

    seg_slic = jnp.ones((B, S, 1), f32)
    return diff_out, sim, seg_slic, pred_R, pred_T


# speedup vs baseline: 1.0846x; 1.0695x over previous
import functools

import jax
import jax.numpy as jnp
from jax import lax
from jax.experimental import pallas as pl
from jax.experimental.pallas import tpu as pltpu

_DIMS_T = (((1,), (1,)), ((), ()))


def _hilo(x):
    hi = x.astype(jnp.bfloat16)
    lo = (x - hi.astype(jnp.float32)).astype(jnp.bfloat16)
    return jnp.concatenate([hi, lo], axis=0)


def _mreg(feat, w0_ref, b0_ref, w1_ref, b1_ref):
    h = jnp.dot(w0_ref[...], feat.astype(jnp.bfloat16),
                preferred_element_type=jnp.float32) + b0_ref[...]
    h = jnp.maximum(h, 0.0)
    return jnp.dot(w1_ref[...], h.astype(jnp.bfloat16),
                   preferred_element_type=jnp.float32) + b1_ref[...]


def _unet(x, oh, wfg_ref, bfg_ref, wo_ref, bo_ref, *, chn):
    fg = jnp.dot(wfg_ref[...], x, preferred_element_type=jnp.float32) + bfg_ref[...]
    feat = jnp.maximum(fg[:chn], 0.0)
    gate = jax.nn.sigmoid(fg[chn:])
    h = (feat * gate).astype(jnp.bfloat16)
    out = jnp.dot(wo_ref[...], h, preferred_element_type=jnp.float32) + bo_ref[...]
    out = jnp.maximum(out, 0.0)
    return lax.dot_general(out.astype(jnp.bfloat16), oh, _DIMS_T,
                           preferred_element_type=jnp.float32)


def _pix_kernel(slic_ref, src_ref, tar_ref, msk_ref,
                mrw0_ref, mrb0_ref, mrw1_ref, mrb1_ref,
                mtw0_ref, mtb0_ref, mtw1_ref, mtb1_ref,
                wfgr_ref, bfgr_ref, wor_ref, bor_ref,
                wfgt_ref, bfgt_ref, wot_ref, bot_ref,
                ps_ref, pra_ref, prt_ref,
                oh_scr, cent_scr, cnt_scr, pred_scr, *, s, chn):
    k = pl.program_id(1)
    src = src_ref[0]
    tar = tar_ref[0]
    tar_neg = (tar[0:1] < 0.0) | (tar[1:2] < 0.0)

    @pl.when(k == 0)
    def _():
        slic = slic_ref[0]
        p_n = src.shape[1]
        seg_v = jnp.where(slic < 0, s, slic)
        seg_a = jnp.where(tar_neg, s, seg_v)
        iota = lax.broadcasted_iota(jnp.int32, (s + 1, p_n), 0)
        oh_a = (seg_a == iota).astype(jnp.bfloat16)
        oh_v = (seg_v == iota).astype(jnp.bfloat16)
        oh_scr[...] = oh_a
        ones = jnp.ones((1, p_n), jnp.float32)
        da = _hilo(jnp.concatenate([src, tar, ones], axis=0))
        dv = _hilo(jnp.concatenate([src, ones], axis=0))
        ra = lax.dot_general(da, oh_a, _DIMS_T, preferred_element_type=jnp.float32)
        rv = lax.dot_general(dv, oh_v, _DIMS_T, preferred_element_type=jnp.float32)
        sums_a = ra[:5] + ra[5:]
        sums_v = rv[:3] + rv[3:]
        cnt = sums_a[4:5]
        cnt_scr[...] = cnt
        cent_scr[...] = sums_a[:4] / jnp.maximum(cnt, 1.0)
        ps_ref[0] = (sums_v[:2] / jnp.maximum(sums_v[2:3], 1.0))[:, :s]

    @pl.when(k == 1)
    def _():
        oh = oh_scr[...]
        g2 = jnp.dot(_hilo(cent_scr[...]), oh, preferred_element_type=jnp.float32)
        g = g2[:4] + g2[4:]
        pm = jnp.concatenate([src - g[:2], tar - g[2:4]], axis=0)
        pm = jnp.where(jnp.logical_not(tar_neg), pm, -1.0)
        x = jnp.concatenate([pm, msk_ref[0]], axis=0).astype(jnp.bfloat16)
        sum_r = _unet(x, oh, wfgr_ref, bfgr_ref, wor_ref, bor_ref, chn=chn)
        feat = sum_r / jnp.maximum(cnt_scr[...], 1.0)
        pred = _mreg(feat, mrw0_ref, mrb0_ref, mrw1_ref, mrb1_ref)
        pred_scr[...] = pred
        pra_ref[0] = pred

    @pl.when(k == 2)
    def _():
        oh = oh_scr[...]
        g2 = jnp.dot(_hilo(pred_scr[...]), oh, preferred_element_type=jnp.float32)
        g = g2[:2] + g2[2:]
        a = g[0:1]
        b = g[1:2]
        rx = src[0:1] * (1.0 + a) + src[1:2] * b
        ry = -src[0:1] * b + src[1:2] * (1.0 + a)
        pm = jnp.concatenate([rx, ry, tar], axis=0)
        pm = jnp.where(jnp.logical_not(tar_neg), pm, -1.0)
        x = jnp.concatenate([pm, msk_ref[0]], axis=0).astype(jnp.bfloat16)
        sum_t = _unet(x, oh, wfgt_ref, bfgt_ref, wot_ref, bot_ref, chn=chn)
        feat = sum_t / jnp.maximum(cnt_scr[...], 1.0)
        prt_ref[0] = _mreg(feat, mtw0_ref, mtb0_ref, mtw1_ref, mtb1_ref)


def _verify_kernel(as_ref, al_ref, w1_ref, b1_ref, w2_ref, b2_ref, w3_ref, b3_ref,
                   g1w_ref, g1b_ref, g2w_ref, g2b_ref, g3w_ref, g3b_ref,
                   wg0_ref, wg1_ref, wu_ref, ub1_ref, uw2_ref, ub2_ref,
                   uw3_ref, ub3_ref, uw4_ref, ub4_ref,
                   d_ref, o_ref, *, s):
    A = as_ref[0]
    L = al_ref[0]
    D0 = (A[:, 0:1] * L[4:5] + A[:, 1:2] * L[5:6] + L[8:9] - A[:, 2:3]
          + A[:, 4:5] * L[0:1] + A[:, 5:6] * L[1:2] + A[:, 8:9] - L[2:3])
    D1 = (A[:, 0:1] * L[6:7] + A[:, 1:2] * L[7:8] + L[9:10] - A[:, 3:4]
          + A[:, 6:7] * L[0:1] + A[:, 7:8] * L[1:2] + A[:, 9:10] - L[3:4])
    P0 = jnp.broadcast_to(A[:, 10:11], (s, s))
    P1 = jnp.broadcast_to(A[:, 11:12], (s, s))
    d_ref[0] = jnp.stack([D0, D1], axis=0)
    x = jnp.stack([D0, D1, P0, P1], axis=0).reshape(4, s * s).astype(jnp.bfloat16)

    h = jnp.maximum(jnp.dot(w1_ref[...], x,
                            preferred_element_type=jnp.float32) + b1_ref[...], 0.0)
    h = jnp.maximum(jnp.dot(w2_ref[...], h.astype(jnp.bfloat16),
                            preferred_element_type=jnp.float32) + b2_ref[...], 0.0)
    h = jnp.maximum(jnp.dot(w3_ref[...], h.astype(jnp.bfloat16),
                            preferred_element_type=jnp.float32) + b3_ref[...], 0.0)
    u = h.astype(jnp.bfloat16)

    u3 = u.reshape(u.shape[0], s, s)
    xg = jnp.concatenate([jnp.max(u3, axis=2), jnp.max(u3, axis=1)], axis=1)
    hg = jnp.maximum(jnp.dot(g1w_ref[...], xg,
                             preferred_element_type=jnp.float32) + g1b_ref[...], 0.0)
    hg = jnp.maximum(jnp.dot(g2w_ref[...], hg.astype(jnp.bfloat16),
                             preferred_element_type=jnp.float32) + g2b_ref[...], 0.0)
    hg = jnp.maximum(jnp.dot(g3w_ref[...], hg.astype(jnp.bfloat16),
                             preferred_element_type=jnp.float32) + g3b_ref[...], 0.0)
    g = hg.astype(jnp.bfloat16)
    pg = jnp.concatenate(
        [jnp.dot(wg0_ref[...], g[:, :s], preferred_element_type=jnp.float32),
         jnp.dot(wg1_ref[...], g[:, s:], preferred_element_type=jnp.float32)],
        axis=1).astype(jnp.bfloat16)

    n = s * s
    p = lax.broadcasted_iota(jnp.int32, (1, n), 1)
    rid = p // s
    cid = p - rid * s
    riota = lax.broadcasted_iota(jnp.int32, (s, n), 0)
    sel = jnp.concatenate([(rid == riota).astype(jnp.bfloat16),
                           (cid == riota).astype(jnp.bfloat16)], axis=0)
    h = jnp.dot(wu_ref[...], u, preferred_element_type=jnp.float32)
    h = h + jnp.dot(pg, sel, preferred_element_type=jnp.float32)
    h = jnp.maximum(h + ub1_ref[...], 0.0)
    h = jnp.maximum(jnp.dot(uw2_ref[...], h.astype(jnp.bfloat16),
                            preferred_element_type=jnp.float32) + ub2_ref[...], 0.0)
    h = jnp.maximum(jnp.dot(uw3_ref[...], h.astype(jnp.bfloat16),
                            preferred_element_type=jnp.float32) + ub3_ref[...], 0.0)
    o_ref[0] = jnp.dot(uw4_ref[...], h.astype(jnp.bfloat16),
                       preferred_element_type=jnp.float32) + ub4_ref[...]


def _wT(w):
    return jnp.transpose(w).astype(jnp.bfloat16)


def _bc(b):
    return b.reshape(-1, 1).astype(jnp.float32)


def kernel(pos_src, pos_tar, mask, slic_map, src_pixel_group, dst_pixel_group,
           unet_r_feat_w, unet_r_feat_b, unet_r_gate_w, unet_r_gate_b,
           unet_r_out_w, unet_r_out_b,
           unet_t_feat_w, unet_t_feat_b, unet_t_gate_w, unet_t_gate_b,
           unet_t_out_w, unet_t_out_b,
           mreg_r_0_w, mreg_r_0_b, mreg_r_1_w, mreg_r_1_b,
           mreg_t_0_w, mreg_t_0_b, mreg_t_1_w, mreg_t_1_b,
           u_pre_0_w, u_pre_0_b, u_pre_1_w, u_pre_1_b, u_pre_2_w, u_pre_2_b,
           u_global_0_w, u_global_0_b, u_global_1_w, u_global_1_b,
           u_global_2_w, u_global_2_b,
           u_post_0_w, u_post_0_b, u_post_1_w, u_post_1_b,
           u_post_2_w, u_post_2_b, u_post_3_w, u_post_3_b):
    B, _, H, W = pos_src.shape
    P = H * W
    S = src_pixel_group.shape[1]
    S1 = S + 1
    SS = S * S
    f32 = jnp.float32

    src = pos_src.reshape(B, 2, P)
    tar = pos_tar.reshape(B, 2, P)
    msk = mask.reshape(B, 1, P)
    slic = slic_map.reshape(B, 1, P).astype(jnp.int32)

    par_arb = pltpu.CompilerParams(dimension_semantics=("parallel", "arbitrary"))
    par = pltpu.CompilerParams(dimension_semantics=("parallel",))

    def unet_w(fw, fb, gw, gb, ow, ob):
        wfg = jnp.transpose(jnp.concatenate([fw, gw], axis=1)).astype(jnp.bfloat16)
        bfg = jnp.concatenate([fb, gb]).reshape(-1, 1).astype(f32)
        return wfg, bfg, _wT(ow), _bc(ob)

    chn = unet_r_feat_w.shape[1]
    wfgr, bfgr, wor, bor = unet_w(unet_r_feat_w, unet_r_feat_b, unet_r_gate_w,
                                  unet_r_gate_b, unet_r_out_w, unet_r_out_b)
    wfgt, bfgt, wot, bot = unet_w(unet_t_feat_w, unet_t_feat_b, unet_t_gate_w,
                                  unet_t_gate_b, unet_t_out_w, unet_t_out_b)
    mrw0, mrw1 = _wT(mreg_r_0_w), _wT(mreg_r_1_w)
    mrb0, mrb1 = _bc(mreg_r_0_b), _bc(mreg_r_1_b)
    mtw0, mtw1 = _wT(mreg_t_0_w), _wT(mreg_t_1_w)
    mtb0, mtb1 = _bc(mreg_t_0_b), _bc(mreg_t_1_b)

    def pspec(c):
        return pl.BlockSpec((1, c, P), lambda i, k: (i, 0, 0))

    def wspec(shape):
        return pl.BlockSpec(shape, lambda i, k: (0, 0))

    pos_sp, pred_ab, pred_t_ab = pl.pallas_call(
        functools.partial(_pix_kernel, s=S, chn=chn),
        out_shape=(jax.ShapeDtypeStruct((B, 2, S), f32),
                   jax.ShapeDtypeStruct((B, 2, S1), f32),
                   jax.ShapeDtypeStruct((B, 2, S1), f32)),
        grid=(B, 3),
        in_specs=[
            pspec(1), pspec(2), pspec(2), pspec(1),
            wspec(mrw0.shape), wspec(mrb0.shape), wspec(mrw1.shape), wspec(mrb1.shape),
            wspec(mtw0.shape), wspec(mtb0.shape), wspec(mtw1.shape), wspec(mtb1.shape),
            wspec(wfgr.shape), wspec(bfgr.shape), wspec(wor.shape), wspec(bor.shape),
            wspec(wfgt.shape), wspec(bfgt.shape), wspec(wot.shape), wspec(bot.shape),
        ],
        out_specs=(pl.BlockSpec((1, 2, S), lambda i, k: (i, 0, 0)),
                   pl.BlockSpec((1, 2, S1), lambda i, k: (i, 0, 0)),
                   pl.BlockSpec((1, 2, S1), lambda i, k: (i, 0, 0))),
        scratch_shapes=[
            pltpu.VMEM((S1, P), jnp.bfloat16),
            pltpu.VMEM((4, S1), f32),
            pltpu.VMEM((1, S1), f32),
            pltpu.VMEM((2, S1), f32),
        ],
        compiler_params=par_arb,
    )(slic, src, tar, msk,
      mrw0, mrb0, mrw1, mrb1, mtw0, mtb0, mtw1, mtb1,
      wfgr, bfgr, wor, bor, wfgt, bfgt, wot, bot)

    a = pred_ab[:, 0, :S]
    b = pred_ab[:, 1, :S]
    pred_R = jnp.stack([jnp.stack([1.0 + a, -b], axis=-1),
                        jnp.stack([b, 1.0 + a], axis=-1)], axis=-2)
    pred_T = jnp.transpose(pred_t_ab, (0, 2, 1))[:, :S][:, :, None, :]

    sm = jnp.mean(src_pixel_group, axis=2)
    dm = jnp.mean(dst_pixel_group, axis=2)
    t0 = pred_t_ab[:, 0, :S]
    t1 = pred_t_ab[:, 1, :S]
    AS = jnp.stack([sm[..., 0], sm[..., 1], dm[..., 0], dm[..., 1],
                    1.0 + a, -b, b, 1.0 + a, t0, t1,
                    pos_sp[:, 0], pos_sp[:, 1]], axis=-1)
    AL = jnp.swapaxes(AS, 1, 2)

    wp1, wp2, wp3 = _wT(u_pre_0_w), _wT(u_pre_1_w), _wT(u_pre_2_w)
    bp1, bp2, bp3 = _bc(u_pre_0_b), _bc(u_pre_1_b), _bc(u_pre_2_b)
    CU = wp3.shape[0]
    w1T = jnp.transpose(u_post_0_w)
    CG = u_global_2_w.shape[1]
    wu = w1T[:, :CU].astype(jnp.bfloat16)
    wg0 = w1T[:, CU:CU + CG].astype(jnp.bfloat16)
    wg1 = w1T[:, CU + CG:CU + 2 * CG].astype(jnp.bfloat16)
    wg_1, wg_2, wg_3 = _wT(u_global_0_w), _wT(u_global_1_w), _wT(u_global_2_w)
    bg_1, bg_2, bg_3 = _bc(u_global_0_b), _bc(u_global_1_b), _bc(u_global_2_b)
    b1c = _bc(u_post_0_b)
    w2t, w3t, w4t = _wT(u_post_1_w), _wT(u_post_2_w), _wT(u_post_3_w)
    b2c, b3c, b4c = _bc(u_post_1_b), _bc(u_post_2_b), _bc(u_post_3_b)

    def ws(shape):
        return pl.BlockSpec(shape, lambda i: (0, 0))

    diff_out, sim = pl.pallas_call(
        functools.partial(_verify_kernel, s=S),
        out_shape=(jax.ShapeDtypeStruct((B, 2, S, S), f32),
                   jax.ShapeDtypeStruct((B, 1, SS), f32)),
        grid=(B,),
        in_specs=[
            pl.BlockSpec((1, S, 12), lambda i: (i, 0, 0)),
            pl.BlockSpec((1, 12, S), lambda i: (i, 0, 0)),
            ws(wp1.shape), ws(bp1.shape), ws(wp2.shape), ws(bp2.shape),
            ws(wp3.shape), ws(bp3.shape),
            ws(wg_1.shape), ws(bg_1.shape), ws(wg_2.shape), ws(bg_2.shape),
            ws(wg_3.shape), ws(bg_3.shape), ws(wg0.shape), ws(wg1.shape),
            ws(wu.shape), ws(b1c.shape), ws(w2t.shape), ws(b2c.shape),
            ws(w3t.shape), ws(b3c.shape), ws(w4t.shape), ws(b4c.shape),
        ],
        out_specs=(pl.BlockSpec((1, 2, S, S), lambda i: (i, 0, 0, 0)),
                   pl.BlockSpec((1, 1, SS), lambda i: (i, 0, 0))),
        compiler_params=par,
    )(AS, AL, wp1, bp1, wp2, bp2, wp3, bp3,
      wg_1, bg_1, wg_2, bg_2, wg_3, bg_3, wg0, wg1,
      wu, b1c, w2t, b2c, w3t, b3c, w4t, b4c)
    sim = sim.reshape(B, S, S)

    seg_slic = jnp.ones((B, S, 1), f32)
    return diff_out, sim, seg_slic, pred_R, pred_T
